# baseline (speedup 1.0000x reference)
import jax
import jax.numpy as jnp
from jax.experimental import pallas as pl
from jax.experimental.pallas import tpu as pltpu

N_ROIS = 5000
IN_CH = 256
ROI = 7
N_SPATIAL = ROI * ROI
FEAT = IN_CH * N_SPATIAL
N_PAD = 184
M_BLK = 384
GRID_M = (N_ROIS + M_BLK - 1) // M_BLK

OFF_REG, OFF_CLS, OFF_FACE, OFF_COLOUR, OFF_MOTION = 0, 128, 160, 168, 176
SMALL0 = 160


def _fused_heads_kernel(x_ref, wta_ref, wtb_ref, b_ref,
                        oc_ref, or_ref, os_ref, wscr):
    @pl.when(pl.program_id(0) == 0)
    def _build_weight_slabs():
        for s in range(N_SPATIAL):
            wscr[s, :, 0:124] = wta_ref[s::N_SPATIAL, :].astype(jnp.bfloat16)
            wscr[s, :, 128:178] = wtb_ref[s::N_SPATIAL, :].astype(jnp.bfloat16)

    acc = b_ref[...].astype(jnp.float32)
    for i in range(ROI):
        for j in range(ROI):
            xs = x_ref[i, j].astype(jnp.bfloat16)
            acc = acc + jnp.dot(xs, wscr[i * ROI + j],
                                preferred_element_type=jnp.float32)
    or_ref[...] = acc[:, OFF_REG:OFF_REG + 124]
    oc_ref[...] = acc[:, OFF_CLS:OFF_CLS + 32].T
    os_ref[...] = acc[:, SMALL0:N_PAD].T


def _fused_matmul(xt, wta, wtb, b_pad):
    return pl.pallas_call(
        _fused_heads_kernel,
        grid=(GRID_M,),
        in_specs=[
            pl.BlockSpec((ROI, ROI, M_BLK, IN_CH), lambda i: (0, 0, i, 0)),
            pl.BlockSpec((FEAT, 124), lambda i: (0, 0)),
            pl.BlockSpec((FEAT, 50), lambda i: (0, 0)),
            pl.BlockSpec((1, N_PAD), lambda i: (0, 0)),
        ],
        out_specs=[
            pl.BlockSpec((32, M_BLK), lambda i: (0, i)),
            pl.BlockSpec((M_BLK, 124), lambda i: (i, 0)),
            pl.BlockSpec((24, M_BLK), lambda i: (0, i)),
        ],
        out_shape=[
            jax.ShapeDtypeStruct((32, N_ROIS), jnp.float32),
            jax.ShapeDtypeStruct((N_ROIS, 124), jnp.float32),
            jax.ShapeDtypeStruct((24, N_ROIS), jnp.float32),
        ],
        scratch_shapes=[
            pltpu.VMEM((N_SPATIAL, IN_CH, N_PAD), jnp.bfloat16),
        ],
    )(xt, wta, wtb, b_pad)


def kernel(x, W_cls, b_cls, W_reg, b_reg, W_face, b_face, W_colour, b_colour, W_motion, b_motion):
    xt = jnp.transpose(x, (2, 3, 0, 1))
    zw = jnp.zeros((5, FEAT), jnp.float32)
    wta = W_reg.T
    wtb = jnp.concatenate(
        [W_cls, W_face, zw[:5], W_colour, zw[:1], W_motion], axis=0).T
    zb = jnp.zeros((6,), jnp.float32)
    b_pad = jnp.concatenate(
        [b_reg, zb[:4], b_cls, b_face, zb[:5], b_colour, zb[:1], b_motion, zb])[None, :]
    oc, orr, osm = _fused_matmul(xt, wta, wtb, b_pad)
    return (
        oc.T,
        orr,
        osm[0:3].T,
        osm[8:15].T,
        osm[16:18].T,
    )


# speedup vs baseline: 1.0787x; 1.0787x over previous
import jax
import jax.numpy as jnp
from jax.experimental import pallas as pl
from jax.experimental.pallas import tpu as pltpu

N_ROIS = 5000
IN_CH = 256
ROI = 7
N_SPATIAL = ROI * ROI
FEAT = IN_CH * N_SPATIAL
N_PAD = 184
M_BLK = 256
GRID_M = (N_ROIS + M_BLK - 1) // M_BLK

OFF_REG, OFF_CLS, OFF_FACE, OFF_COLOUR, OFF_MOTION = 0, 128, 160, 168, 176
SMALL0 = 160


def _fused_heads_kernel(x_ref, wta_ref, wtb_ref, b_ref,
                        oc_ref, or_ref, of_ref, ol_ref, om_ref, wscr):
    @pl.when(pl.program_id(0) == 0)
    def _build_weight_slabs():
        for s in range(N_SPATIAL):
            wscr[s, :, 0:124] = wta_ref[s::N_SPATIAL, :].astype(jnp.bfloat16)
            wscr[s, :, 128:178] = wtb_ref[s::N_SPATIAL, :].astype(jnp.bfloat16)

    acc = b_ref[...].astype(jnp.float32)
    for i in range(ROI):
        for j in range(ROI):
            xs = x_ref[i, j].astype(jnp.bfloat16)
            acc = acc + jnp.dot(xs, wscr[i * ROI + j],
                                preferred_element_type=jnp.float32)
    or_ref[...] = acc[:, OFF_REG:OFF_REG + 124]
    oc_ref[...] = acc[:, OFF_CLS:OFF_CLS + 32].T
    ts = acc[:, SMALL0:N_PAD].T
    of_ref[...] = ts[0:3]
    ol_ref[...] = ts[8:15]
    om_ref[...] = ts[16:18]


def _fused_matmul(xt, wta, wtb, b_pad):
    return pl.pallas_call(
        _fused_heads_kernel,
        grid=(GRID_M,),
        in_specs=[
            pl.BlockSpec((ROI, ROI, M_BLK, IN_CH), lambda i: (0, 0, i, 0)),
            pl.BlockSpec((FEAT, 124), lambda i: (0, 0)),
            pl.BlockSpec((FEAT, 50), lambda i: (0, 0)),
            pl.BlockSpec((1, N_PAD), lambda i: (0, 0)),
        ],
        out_specs=[
            pl.BlockSpec((32, M_BLK), lambda i: (0, i)),
            pl.BlockSpec((M_BLK, 124), lambda i: (i, 0)),
            pl.BlockSpec((3, M_BLK), lambda i: (0, i)),
            pl.BlockSpec((7, M_BLK), lambda i: (0, i)),
            pl.BlockSpec((2, M_BLK), lambda i: (0, i)),
        ],
        out_shape=[
            jax.ShapeDtypeStruct((32, N_ROIS), jnp.float32),
            jax.ShapeDtypeStruct((N_ROIS, 124), jnp.float32),
            jax.ShapeDtypeStruct((3, N_ROIS), jnp.float32),
            jax.ShapeDtypeStruct((7, N_ROIS), jnp.float32),
            jax.ShapeDtypeStruct((2, N_ROIS), jnp.float32),
        ],
        scratch_shapes=[
            pltpu.VMEM((N_SPATIAL, IN_CH, N_PAD), jnp.bfloat16),
        ],
    )(xt, wta, wtb, b_pad)


def kernel(x, W_cls, b_cls, W_reg, b_reg, W_face, b_face, W_colour, b_colour, W_motion, b_motion):
    xt = jnp.transpose(x, (2, 3, 0, 1))
    zw = jnp.zeros((5, FEAT), jnp.float32)
    wta = W_reg.T
    wtb = jnp.concatenate(
        [W_cls, W_face, zw[:5], W_colour, zw[:1], W_motion], axis=0).T
    zb = jnp.zeros((6,), jnp.float32)
    b_pad = jnp.concatenate(
        [b_reg, zb[:4], b_cls, b_face, zb[:5], b_colour, zb[:1], b_motion, zb])[None, :]
    oc, orr, of, ol, om = _fused_matmul(xt, wta, wtb, b_pad)
    return (oc.T, orr, of.T, ol.T, om.T)


# speedup vs baseline: 1.1690x; 1.0836x over previous
import jax
import jax.numpy as jnp
from jax.experimental import pallas as pl
from jax.experimental.pallas import tpu as pltpu

N_ROIS = 5000
IN_CH = 256
ROI = 7
N_SPATIAL = ROI * ROI
FEAT = IN_CH * N_SPATIAL
N_PAD = 184
M_BLK = 256
GRID_M = (N_ROIS + M_BLK - 1) // M_BLK

OFF_REG, OFF_CLS, OFF_FACE, OFF_COLOUR, OFF_MOTION = 0, 128, 160, 168, 176
SMALL0 = 160


def _fused_heads_kernel(x_ref, wr_ref, wc_ref, wf_ref, wl_ref, wm_ref, b_ref,
                        oc_ref, or_ref, of_ref, ol_ref, om_ref,
                        nr_scr, nc_scr, nf_scr, nl_scr, nm_scr,
                        wtmpa, wtmpb, wscr, sem):
    @pl.when(pl.program_id(0) == 0)
    def _build_weight_slabs():
        half = N_SPATIAL * 128
        ck = 896
        for h2 in range(2):
            base = h2 * half
            copies = [
                pltpu.make_async_copy(wr_ref.at[:, base:base + half], nr_scr, sem),
                pltpu.make_async_copy(wc_ref.at[:, base:base + half], nc_scr, sem),
                pltpu.make_async_copy(wf_ref.at[:, base:base + half], nf_scr, sem),
                pltpu.make_async_copy(wl_ref.at[:, base:base + half], nl_scr, sem),
                pltpu.make_async_copy(wm_ref.at[:, base:base + half], nm_scr, sem),
            ]
            for c in copies:
                c.start()
            for c in copies:
                c.wait()
            for g in range(0, half, ck):
                wtmpa[g:g + ck, 0:124] = nr_scr[:, g:g + ck].T
                wtmpb[g:g + ck, 0:32] = nc_scr[:, g:g + ck].T
                wtmpb[g:g + ck, 32:35] = nf_scr[:, g:g + ck].T
                wtmpb[g:g + ck, 40:47] = nl_scr[:, g:g + ck].T
                wtmpb[g:g + ck, 48:50] = nm_scr[:, g:g + ck].T
            c0 = h2 * 128
            for s in range(N_SPATIAL):
                wscr[s, c0:c0 + 128, 0:124] = (
                    wtmpa[s::N_SPATIAL, 0:124].astype(jnp.bfloat16))
                wscr[s, c0:c0 + 128, 128:178] = (
                    wtmpb[s::N_SPATIAL, 0:50].astype(jnp.bfloat16))

    acc = b_ref[...].astype(jnp.float32)
    for i in range(ROI):
        for j in range(ROI):
            xs = x_ref[i, j].astype(jnp.bfloat16)
            acc = acc + jnp.dot(xs, wscr[i * ROI + j],
                                preferred_element_type=jnp.float32)
    or_ref[...] = acc[:, OFF_REG:OFF_REG + 124]
    oc_ref[...] = acc[:, OFF_CLS:OFF_CLS + 32].T
    ts = acc[:, SMALL0:N_PAD].T
    of_ref[...] = ts[0:3]
    ol_ref[...] = ts[8:15]
    om_ref[...] = ts[16:18]


def _fused_matmul(xt, ws, b_pad):
    return pl.pallas_call(
        _fused_heads_kernel,
        grid=(GRID_M,),
        in_specs=[
            pl.BlockSpec((ROI, ROI, M_BLK, IN_CH), lambda i: (0, 0, i, 0)),
            pl.BlockSpec(memory_space=pltpu.MemorySpace.HBM),
            pl.BlockSpec(memory_space=pltpu.MemorySpace.HBM),
            pl.BlockSpec(memory_space=pltpu.MemorySpace.HBM),
            pl.BlockSpec(memory_space=pltpu.MemorySpace.HBM),
            pl.BlockSpec(memory_space=pltpu.MemorySpace.HBM),
            pl.BlockSpec((1, N_PAD), lambda i: (0, 0)),
        ],
        out_specs=[
            pl.BlockSpec((32, M_BLK), lambda i: (0, i)),
            pl.BlockSpec((M_BLK, 124), lambda i: (i, 0)),
            pl.BlockSpec((3, M_BLK), lambda i: (0, i)),
            pl.BlockSpec((7, M_BLK), lambda i: (0, i)),
            pl.BlockSpec((2, M_BLK), lambda i: (0, i)),
        ],
        out_shape=[
            jax.ShapeDtypeStruct((32, N_ROIS), jnp.float32),
            jax.ShapeDtypeStruct((N_ROIS, 124), jnp.float32),
            jax.ShapeDtypeStruct((3, N_ROIS), jnp.float32),
            jax.ShapeDtypeStruct((7, N_ROIS), jnp.float32),
            jax.ShapeDtypeStruct((2, N_ROIS), jnp.float32),
        ],
        scratch_shapes=[
            pltpu.VMEM((124, N_SPATIAL * 128), jnp.float32),
            pltpu.VMEM((32, N_SPATIAL * 128), jnp.float32),
            pltpu.VMEM((3, N_SPATIAL * 128), jnp.float32),
            pltpu.VMEM((7, N_SPATIAL * 128), jnp.float32),
            pltpu.VMEM((2, N_SPATIAL * 128), jnp.float32),
            pltpu.VMEM((N_SPATIAL * 128, 124), jnp.float32),
            pltpu.VMEM((N_SPATIAL * 128, 50), jnp.float32),
            pltpu.VMEM((N_SPATIAL, IN_CH, N_PAD), jnp.bfloat16),
            pltpu.SemaphoreType.DMA,
        ],
    )(xt, *ws, b_pad)


def kernel(x, W_cls, b_cls, W_reg, b_reg, W_face, b_face, W_colour, b_colour, W_motion, b_motion):
    xt = jnp.transpose(x, (2, 3, 0, 1))
    zb = jnp.zeros((6,), jnp.float32)
    b_pad = jnp.concatenate(
        [b_reg, zb[:4], b_cls, b_face, zb[:5], b_colour, zb[:1], b_motion, zb])[None, :]
    oc, orr, of, ol, om = _fused_matmul(
        xt, (W_reg, W_cls, W_face, W_colour, W_motion), b_pad)
    return (oc.T, orr, of.T, ol.T, om.T)
